# baseline (speedup 1.0000x reference)
import functools

import jax
import jax.numpy as jnp
from jax import lax
from jax.experimental import pallas as pl
from jax.experimental.pallas import tpu as pltpu
from jax.experimental.pallas import tpu_sc as plsc

N = 10000
E = 320000
D = 128
DA = D + 16

NC = 2
NS = 16
NW = NC * NS
EPT = E // NW
NPAD = 10240
RPT = NPAD // NS
PKB = 14
PKM = (1 << PKB) - 1
L = 16

CH_D, STEPS_D = 128, 80
CH_A, STEPS_A = 80, 125

_SC_MESH = plsc.VectorSubcoreMesh(core_axis_name="c", subcore_axis_name="s",
                                  num_cores=NC, num_subcores=NS)


def _make_sc_agg(W, CH, STEPS):
    NTR = RPT // CH

    @functools.partial(
        pl.kernel,
        out_type=jax.ShapeDtypeStruct((NC, NPAD, W), jnp.float32),
        mesh=_SC_MESH,
        scratch_types=[
            pltpu.VMEM((STEPS, CH), jnp.int32),
            pltpu.VMEM((CH,), jnp.int32),
            pltpu.VMEM((CH,), jnp.int32),
            pltpu.VMEM((CH, W), jnp.float32),
            pltpu.VMEM_SHARED((NPAD, W), jnp.float32),
            pltpu.SemaphoreType.DMA,
        ],
        compiler_params=pltpu.CompilerParams(use_tc_tiling_on_sc=(W == D)),
    )
    def _agg(h_hbm, pk_hbm, zrow_hbm, agg_out, pkv, srcc, dstc, rows, acc,
             sem):
        cid = lax.axis_index("c")
        sid = lax.axis_index("s")
        wid = cid * NS + sid
        row0 = sid * RPT

        pltpu.sync_copy(pk_hbm.at[wid], pkv)
        pltpu.sync_copy(zrow_hbm, rows)

        def zero_body(kk, carry):
            pltpu.sync_copy(rows, acc.at[pl.ds(row0 + kk * CH, CH)])
            return carry

        lax.fori_loop(0, NTR, zero_body, 0)
        plsc.subcore_barrier()

        def step(j, carry):
            for k in range(CH // L):
                v = pkv[j, pl.ds(k * L, L)]
                srcc[pl.ds(k * L, L)] = jax.lax.shift_right_logical(v, PKB)
                dstc[pl.ds(k * L, L)] = jax.lax.bitwise_and(v, PKM)
            pltpu.async_copy(h_hbm.at[srcc], rows, sem).wait()
            pltpu.sync_copy(rows, acc.at[dstc], add=True)
            return carry

        lax.fori_loop(0, STEPS, step, 0)
        plsc.subcore_barrier()

        def read_body(kk, carry):
            pltpu.sync_copy(acc.at[pl.ds(row0 + kk * CH, CH)], rows)
            pltpu.sync_copy(rows,
                            agg_out.at[cid].at[pl.ds(row0 + kk * CH, CH)])
            return carry

        lax.fori_loop(0, NTR, read_body, 0)

    return _agg


_sc_agg = _make_sc_agg(D, CH_D, STEPS_D)
_sc_agg_aug = _make_sc_agg(DA, CH_A, STEPS_A)


R = 2000


def _tc_pre_body(x_ref, wpT_ref, out_ref):
    out_ref[:, :D] = jnp.dot(x_ref[...], wpT_ref[...],
                             preferred_element_type=jnp.float32)
    out_ref[:, D:] = jnp.ones((R, DA - D), jnp.float32)


def _tc_pre(x, wpT):
    return pl.pallas_call(
        _tc_pre_body,
        grid=(N // R,),
        in_specs=[pl.BlockSpec((R, D), lambda i: (i, 0)),
                  pl.BlockSpec((D, D), lambda i: (0, 0))],
        out_specs=pl.BlockSpec((R, DA), lambda i: (i, 0)),
        out_shape=jax.ShapeDtypeStruct((N, DA), jnp.float32),
    )(x, wpT)


def _tc_layer_body(aggp_ref, degp_ref, h_ref, res_ref, wlT_ref, wrT_ref,
                   bl_ref, out_ref, *, mode):
    s = aggp_ref[0] + aggp_ref[1]
    agg = s[:, :D]
    deg = degp_ref[0][:, D:D + 1] + degp_ref[1][:, D:D + 1]
    agg = agg / jnp.maximum(deg, 1.0)
    h = h_ref[...][:, :D]
    out = (jnp.dot(agg, wlT_ref[...], preferred_element_type=jnp.float32)
           + bl_ref[...]
           + jnp.dot(h, wrT_ref[...], preferred_element_type=jnp.float32))
    if mode == "relu_res":
        out = jnp.maximum(out, 0.0) + res_ref[...]
    else:
        norm = jnp.sqrt(jnp.sum(out * out, axis=1, keepdims=True))
        out = out / jnp.maximum(norm, 1e-12)
    out_ref[...] = out


def _tc_layer(aggp, degp, h, res, wlT, wrT, bl, mode):
    wa = aggp.shape[-1]
    wh = h.shape[-1]
    return pl.pallas_call(
        functools.partial(_tc_layer_body, mode=mode),
        grid=(N // R,),
        in_specs=[pl.BlockSpec((NC, R, wa), lambda i: (0, i, 0)),
                  pl.BlockSpec((NC, R, DA), lambda i: (0, i, 0)),
                  pl.BlockSpec((R, wh), lambda i: (i, 0)),
                  pl.BlockSpec((R, D), lambda i: (i, 0)),
                  pl.BlockSpec((D, D), lambda i: (0, 0)),
                  pl.BlockSpec((D, D), lambda i: (0, 0)),
                  pl.BlockSpec((1, D), lambda i: (0, 0))],
        out_specs=pl.BlockSpec((R, D), lambda i: (i, 0)),
        out_shape=jax.ShapeDtypeStruct((N, D), jnp.float32),
    )(aggp, degp, h, res, wlT, wrT, bl)


def _pack_edges(edge_index, steps, ch):
    packed = ((edge_index[0] << PKB) | edge_index[1]).reshape(NW, EPT)
    if steps * ch > EPT:
        pad = jnp.full((NW, steps * ch - EPT), N, jnp.int32)
        packed = jnp.concatenate([packed, pad], axis=1)
    return packed.reshape(NW, steps, ch)


def kernel(x, edge_index, Wp, Wl1, bl1, Wr1, Wl2, bl2, Wr2, Wl3, bl3, Wr3):
    packed_d = _pack_edges(edge_index, STEPS_D, CH_D)
    packed_a = _pack_edges(edge_index, STEPS_A, CH_A)
    zrow = jnp.zeros((CH_D, D), jnp.float32)
    zrow_a = jnp.zeros((CH_A, DA), jnp.float32)

    h0 = _tc_pre(x, Wp.T)
    aggp1 = _sc_agg_aug(h0, packed_a, zrow_a)
    h1 = _tc_layer(aggp1, aggp1, h0, x, Wl1.T, Wr1.T, bl1.reshape(1, D),
                   "relu_res")
    aggp2 = _sc_agg(h1, packed_d, zrow)
    h2 = _tc_layer(aggp2, aggp1, h1, h1, Wl2.T, Wr2.T, bl2.reshape(1, D),
                   "relu_res")
    aggp3 = _sc_agg(h2, packed_d, zrow)
    return _tc_layer(aggp3, aggp1, h2, h2, Wl3.T, Wr3.T, bl3.reshape(1, D),
                     "norm")


# speedup vs baseline: 1.4056x; 1.4056x over previous
import functools

import jax
import jax.numpy as jnp
from jax import lax
from jax.experimental import pallas as pl
from jax.experimental.pallas import tpu as pltpu
from jax.experimental.pallas import tpu_sc as plsc

N = 10000
E = 320000
D = 128
DA = D + 16

NC = 2
NS = 16
NW = NC * NS
EPT = E // NW
NPAD = 10240
RPT = NPAD // NS
PKB = 14
PKM = (1 << PKB) - 1
L = 16

CH_D, STEPS_D = 64, 157
CH_A, STEPS_A = 80, 125

_SC_MESH = plsc.VectorSubcoreMesh(core_axis_name="c", subcore_axis_name="s",
                                  num_cores=NC, num_subcores=NS)


def _make_sc_agg(W, CH, STEPS):
    NTR = RPT // CH

    @functools.partial(
        pl.kernel,
        out_type=jax.ShapeDtypeStruct((NC, NPAD, W), jnp.float32),
        mesh=_SC_MESH,
        scratch_types=[
            pltpu.VMEM((STEPS, CH), jnp.int32),
            pltpu.VMEM((CH,), jnp.int32),
            pltpu.VMEM((CH,), jnp.int32),
            pltpu.VMEM((CH, W), jnp.float32),
            pltpu.VMEM_SHARED((NPAD, W), jnp.float32),
            pltpu.SemaphoreType.DMA,
        ],
        compiler_params=pltpu.CompilerParams(use_tc_tiling_on_sc=(W == D)),
    )
    def _agg(h_hbm, pk_hbm, zrow_hbm, agg_out, pkv, srcc, dstc, rows, acc,
             sem):
        cid = lax.axis_index("c")
        sid = lax.axis_index("s")
        wid = cid * NS + sid
        row0 = sid * RPT

        pltpu.sync_copy(pk_hbm.at[wid], pkv)
        pltpu.sync_copy(zrow_hbm, rows)

        def zero_body(kk, carry):
            pltpu.sync_copy(rows, acc.at[pl.ds(row0 + kk * CH, CH)])
            return carry

        lax.fori_loop(0, NTR, zero_body, 0)
        plsc.subcore_barrier()

        def step(j, carry):
            for k in range(CH // L):
                v = pkv[j, pl.ds(k * L, L)]
                srcc[pl.ds(k * L, L)] = jax.lax.shift_right_logical(v, PKB)
                dstc[pl.ds(k * L, L)] = jax.lax.bitwise_and(v, PKM)
            pltpu.async_copy(h_hbm.at[srcc], rows, sem).wait()
            pltpu.sync_copy(rows, acc.at[dstc], add=True)
            return carry

        lax.fori_loop(0, STEPS, step, 0)
        plsc.subcore_barrier()

        def read_body(kk, carry):
            pltpu.sync_copy(acc.at[pl.ds(row0 + kk * CH, CH)], rows)
            pltpu.sync_copy(rows,
                            agg_out.at[cid].at[pl.ds(row0 + kk * CH, CH)])
            return carry

        lax.fori_loop(0, NTR, read_body, 0)

    return _agg


_sc_agg = _make_sc_agg(D, CH_D, STEPS_D)
_sc_agg_aug = _make_sc_agg(DA, CH_A, STEPS_A)


R = 2000


def _tc_pre_body(x_ref, wpT_ref, out_ref):
    out_ref[:, :D] = jnp.dot(x_ref[...], wpT_ref[...],
                             preferred_element_type=jnp.float32)
    out_ref[:, D:] = jnp.ones((R, DA - D), jnp.float32)


def _tc_pre(x, wpT):
    return pl.pallas_call(
        _tc_pre_body,
        grid=(N // R,),
        in_specs=[pl.BlockSpec((R, D), lambda i: (i, 0)),
                  pl.BlockSpec((D, D), lambda i: (0, 0))],
        out_specs=pl.BlockSpec((R, DA), lambda i: (i, 0)),
        out_shape=jax.ShapeDtypeStruct((N, DA), jnp.float32),
    )(x, wpT)


def _tc_layer_body(aggp_ref, degp_ref, h_ref, res_ref, wlT_ref, wrT_ref,
                   bl_ref, out_ref, *, mode):
    s = aggp_ref[0] + aggp_ref[1]
    agg = s[:, :D]
    deg = degp_ref[0][:, D:D + 1] + degp_ref[1][:, D:D + 1]
    agg = agg / jnp.maximum(deg, 1.0)
    h = h_ref[...][:, :D]
    out = (jnp.dot(agg, wlT_ref[...], preferred_element_type=jnp.float32)
           + bl_ref[...]
           + jnp.dot(h, wrT_ref[...], preferred_element_type=jnp.float32))
    if mode == "relu_res":
        out = jnp.maximum(out, 0.0) + res_ref[...]
    else:
        norm = jnp.sqrt(jnp.sum(out * out, axis=1, keepdims=True))
        out = out / jnp.maximum(norm, 1e-12)
    out_ref[...] = out


def _tc_layer(aggp, degp, h, res, wlT, wrT, bl, mode):
    wa = aggp.shape[-1]
    wh = h.shape[-1]
    return pl.pallas_call(
        functools.partial(_tc_layer_body, mode=mode),
        grid=(N // R,),
        in_specs=[pl.BlockSpec((NC, R, wa), lambda i: (0, i, 0)),
                  pl.BlockSpec((NC, R, DA), lambda i: (0, i, 0)),
                  pl.BlockSpec((R, wh), lambda i: (i, 0)),
                  pl.BlockSpec((R, D), lambda i: (i, 0)),
                  pl.BlockSpec((D, D), lambda i: (0, 0)),
                  pl.BlockSpec((D, D), lambda i: (0, 0)),
                  pl.BlockSpec((1, D), lambda i: (0, 0))],
        out_specs=pl.BlockSpec((R, D), lambda i: (i, 0)),
        out_shape=jax.ShapeDtypeStruct((N, D), jnp.float32),
    )(aggp, degp, h, res, wlT, wrT, bl)


def _pack_edges(edge_index, steps, ch):
    packed = ((edge_index[0] << PKB) | edge_index[1]).reshape(NW, EPT)
    if steps * ch > EPT:
        pad = jnp.full((NW, steps * ch - EPT), N, jnp.int32)
        packed = jnp.concatenate([packed, pad], axis=1)
    return packed.reshape(NW, steps, ch)


def kernel(x, edge_index, Wp, Wl1, bl1, Wr1, Wl2, bl2, Wr2, Wl3, bl3, Wr3):
    packed_d = _pack_edges(edge_index, STEPS_D, CH_D)
    packed_a = _pack_edges(edge_index, STEPS_A, CH_A)
    zrow = jnp.zeros((CH_D, D), jnp.float32)
    zrow_a = jnp.zeros((CH_A, DA), jnp.float32)

    h0 = _tc_pre(x, Wp.T)
    aggp1 = _sc_agg_aug(h0, packed_a, zrow_a)
    h1 = _tc_layer(aggp1, aggp1, h0, x, Wl1.T, Wr1.T, bl1.reshape(1, D),
                   "relu_res")
    aggp2 = _sc_agg(h1, packed_d, zrow)
    h2 = _tc_layer(aggp2, aggp1, h1, h1, Wl2.T, Wr2.T, bl2.reshape(1, D),
                   "relu_res")
    aggp3 = _sc_agg(h2, packed_d, zrow)
    return _tc_layer(aggp3, aggp1, h2, h2, Wl3.T, Wr3.T, bl3.reshape(1, D),
                     "norm")


# speedup vs baseline: 1.6540x; 1.1767x over previous
import functools

import jax
import jax.numpy as jnp
from jax import lax
from jax.experimental import pallas as pl
from jax.experimental.pallas import tpu as pltpu
from jax.experimental.pallas import tpu_sc as plsc

N = 10000
E = 320000
D = 128
DA = D + 16

NC = 2
NS = 16
NW = NC * NS
EPT = E // NW
CH = 80
STEPS = EPT // CH
NPAD = 10240
RPT = NPAD // NS
NTR = RPT // CH
PKB = 14
PKM = (1 << PKB) - 1
L = 16

_SC_MESH = plsc.VectorSubcoreMesh(core_axis_name="c", subcore_axis_name="s",
                                  num_cores=NC, num_subcores=NS)


def _make_sc_agg(W):
    @functools.partial(
        pl.kernel,
        out_type=jax.ShapeDtypeStruct((NC, NPAD, W), jnp.float32),
        mesh=_SC_MESH,
        scratch_types=[
            pltpu.VMEM((STEPS, CH), jnp.int32),
            pltpu.VMEM((CH,), jnp.int32),
            pltpu.VMEM((CH,), jnp.int32),
            pltpu.VMEM((CH, W), jnp.float32),
            pltpu.VMEM_SHARED((NPAD, W), jnp.float32),
            pltpu.SemaphoreType.DMA,
        ],
        compiler_params=pltpu.CompilerParams(use_tc_tiling_on_sc=(W == D)),
    )
    def _agg(h_hbm, pk_hbm, zrow_hbm, agg_out, pkv, srcc, dstc, rows, acc,
             sem):
        cid = lax.axis_index("c")
        sid = lax.axis_index("s")
        wid = cid * NS + sid
        row0 = sid * RPT

        pltpu.sync_copy(pk_hbm.at[wid], pkv)
        pltpu.sync_copy(zrow_hbm, rows)

        def zero_body(kk, carry):
            pltpu.sync_copy(rows, acc.at[pl.ds(row0 + kk * CH, CH)])
            return carry

        lax.fori_loop(0, NTR, zero_body, 0)
        plsc.subcore_barrier()

        def step(j, carry):
            for k in range(CH // L):
                v = pkv[j, pl.ds(k * L, L)]
                srcc[pl.ds(k * L, L)] = jax.lax.shift_right_logical(v, PKB)
                dstc[pl.ds(k * L, L)] = jax.lax.bitwise_and(v, PKM)
            pltpu.async_copy(h_hbm.at[srcc], rows, sem).wait()
            pltpu.sync_copy(rows, acc.at[dstc], add=True)
            return carry

        lax.fori_loop(0, STEPS, step, 0)
        plsc.subcore_barrier()

        def read_body(kk, carry):
            pltpu.sync_copy(acc.at[pl.ds(row0 + kk * CH, CH)], rows)
            pltpu.sync_copy(rows,
                            agg_out.at[cid].at[pl.ds(row0 + kk * CH, CH)])
            return carry

        lax.fori_loop(0, NTR, read_body, 0)

    return _agg


_sc_agg = _make_sc_agg(D)
_sc_agg_aug = _make_sc_agg(DA)


R = 2000


def _tc_pre_body(x_ref, wpT_ref, out_ref):
    out_ref[:, :D] = jnp.dot(x_ref[...], wpT_ref[...],
                             preferred_element_type=jnp.float32)
    out_ref[:, D:] = jnp.ones((R, DA - D), jnp.float32)


def _tc_pre(x, wpT):
    return pl.pallas_call(
        _tc_pre_body,
        grid=(N // R,),
        in_specs=[pl.BlockSpec((R, D), lambda i: (i, 0)),
                  pl.BlockSpec((D, D), lambda i: (0, 0))],
        out_specs=pl.BlockSpec((R, DA), lambda i: (i, 0)),
        out_shape=jax.ShapeDtypeStruct((N, DA), jnp.float32),
    )(x, wpT)


def _tc_layer_body(aggp_ref, degp_ref, h_ref, res_ref, wlT_ref, wrT_ref,
                   bl_ref, out_ref, *, mode):
    s = aggp_ref[0] + aggp_ref[1]
    agg = s[:, :D]
    deg = degp_ref[0][:, D:D + 1] + degp_ref[1][:, D:D + 1]
    agg = agg / jnp.maximum(deg, 1.0)
    h = h_ref[...][:, :D]
    out = (jnp.dot(agg, wlT_ref[...], preferred_element_type=jnp.float32)
           + bl_ref[...]
           + jnp.dot(h, wrT_ref[...], preferred_element_type=jnp.float32))
    if mode == "relu_res":
        out = jnp.maximum(out, 0.0) + res_ref[...]
    else:
        norm = jnp.sqrt(jnp.sum(out * out, axis=1, keepdims=True))
        out = out / jnp.maximum(norm, 1e-12)
    out_ref[...] = out


def _tc_layer(aggp, degp, h, res, wlT, wrT, bl, mode):
    wa = aggp.shape[-1]
    wh = h.shape[-1]
    return pl.pallas_call(
        functools.partial(_tc_layer_body, mode=mode),
        grid=(N // R,),
        in_specs=[pl.BlockSpec((NC, R, wa), lambda i: (0, i, 0)),
                  pl.BlockSpec((NC, R, DA), lambda i: (0, i, 0)),
                  pl.BlockSpec((R, wh), lambda i: (i, 0)),
                  pl.BlockSpec((R, D), lambda i: (i, 0)),
                  pl.BlockSpec((D, D), lambda i: (0, 0)),
                  pl.BlockSpec((D, D), lambda i: (0, 0)),
                  pl.BlockSpec((1, D), lambda i: (0, 0))],
        out_specs=pl.BlockSpec((R, D), lambda i: (i, 0)),
        out_shape=jax.ShapeDtypeStruct((N, D), jnp.float32),
    )(aggp, degp, h, res, wlT, wrT, bl)


def kernel(x, edge_index, Wp, Wl1, bl1, Wr1, Wl2, bl2, Wr2, Wl3, bl3, Wr3):
    packed = ((edge_index[0] << PKB) | edge_index[1]).reshape(NW, STEPS, CH)
    zrow = jnp.zeros((CH, D), jnp.float32)
    zrow_a = jnp.zeros((CH, DA), jnp.float32)

    h0 = _tc_pre(x, Wp.T)
    aggp1 = _sc_agg_aug(h0, packed, zrow_a)
    h1 = _tc_layer(aggp1, aggp1, h0, x, Wl1.T, Wr1.T, bl1.reshape(1, D),
                   "relu_res")
    aggp2 = _sc_agg(h1, packed, zrow)
    h2 = _tc_layer(aggp2, aggp1, h1, h1, Wl2.T, Wr2.T, bl2.reshape(1, D),
                   "relu_res")
    aggp3 = _sc_agg(h2, packed, zrow)
    return _tc_layer(aggp3, aggp1, h2, h2, Wl3.T, Wr3.T, bl3.reshape(1, D),
                     "norm")
